# baseline (speedup 1.0000x reference)
import functools

import jax
import jax.numpy as jnp
from jax import lax
from jax.experimental import pallas as pl
from jax.experimental.pallas import tpu as pltpu
from jax.experimental.pallas import tpu_sc as plsc

N = 10000
E = 320000
F_IN = 128
HID = 512
H1 = 8
OUT = 2

NC = 2
NS = 16
NW = NC * NS
EDGES_PER_W = E // NW
CHUNK = 80
NCHUNK = EDGES_PER_W // CHUNK
ROWS_PER_S = N // NS

_LEAK = 0.2


def _leaky(v):
    return jnp.where(v > 0, v, _LEAK * v)


def _perm(v, idx):
    dnums = lax.GatherDimensionNumbers(
        offset_dims=(), collapsed_slice_dims=(0,), start_index_map=(0,))
    return lax.gather(v, idx[:, None], dnums, (1,),
                      mode=lax.GatherScatterMode.PROMISE_IN_BOUNDS)


def _tc1_body(x_ref, bg_ref, bd_ref, gs_ref, d_ref):
    xv = x_ref[...]
    gs = jnp.dot(xv, bg_ref[...], preferred_element_type=jnp.float32)
    gs_ref[...] = gs
    a1d = jnp.dot(xv, bd_ref[...], preferred_element_type=jnp.float32)
    a1s = gs[:, 0:8]
    gmax = jnp.max(a1s, axis=0, keepdims=True)
    m = gmax + a1d
    d_ref[...] = jnp.concatenate([a1d, _leaky(m)], axis=1)


def _tc2a_body(pd_ref, inv_ref):
    inv_ref[...] = 1.0 / (jnp.sum(pd_ref[...], axis=0) + 1e-16)


def _tc2_body(p_ref, inv_ref, cs_ref, cd_ref, a2s_w_ref, a2d_w_ref,
              g2_ref, d2_ref):
    acc = p_ref[0] + p_ref[1]
    inv = inv_ref[...]
    rsel = (lax.broadcasted_iota(jnp.int32, (8, 16), 0)
            == (lax.broadcasted_iota(jnp.int32, (8, 16), 1) >> 1)
            ).astype(jnp.float32)
    inv16 = jnp.dot(inv, rsel, preferred_element_type=jnp.float32)
    zsi = acc[:, 0:16] * inv16
    zdi = acc[:, 16:32] * inv16
    lane = lax.broadcasted_iota(jnp.int32, (16, 2), 0)
    col = lax.broadcasted_iota(jnp.int32, (16, 2), 1)
    sel = (lane % 2 == col).astype(jnp.float32)
    xs2 = jnp.dot(zsi, sel, preferred_element_type=jnp.float32) + cs_ref[...]
    xd2 = jnp.dot(zdi, sel, preferred_element_type=jnp.float32) + cd_ref[...]
    a2s = jnp.dot(xs2, a2s_w_ref[...], preferred_element_type=jnp.float32)
    a2d = jnp.dot(xd2, a2d_w_ref[...], preferred_element_type=jnp.float32)
    m2 = _leaky(jnp.max(a2s, axis=0, keepdims=True) + a2d)
    ones = jnp.ones_like(a2s)
    zeros4 = jnp.zeros((xs2.shape[0], 4), jnp.float32)
    g2_ref[...] = jnp.concatenate([a2s, ones, xs2, zeros4], axis=1)
    d2_ref[...] = jnp.concatenate([a2d, m2, zeros4, jnp.zeros_like(xs2)], axis=1)


def _tc3_body(p2_ref, b2_ref, out_ref):
    acc2 = p2_ref[0] + p2_ref[1]
    denom = acc2[:, 0:1]
    num = acc2[:, 1:3]
    out_ref[...] = num / (denom + 1e-16) + b2_ref[...]


_MESH = plsc.VectorSubcoreMesh(core_axis_name="c", subcore_axis_name="s",
                               num_cores=NC, num_subcores=NS)


def _sc_edge_pipeline(compute_chunk):
    def body(idx_hbm, g_hbm, d_hbm, z_hbm, out_hbm,
             acc_sh,
             idx_a, idx_b, sd_a, sd_b,
             g_a, g_b, d_a, d_b, m_a, m_b,
             sg_a, sg_b, sdm_a, sdm_b, ss_a, ss_b):
        c = lax.axis_index("c")
        s = lax.axis_index("s")
        wid = s * NC + c
        row0 = s * ROWS_PER_S

        idxv = (idx_a, idx_b)
        sdst = (sd_a, sd_b)
        gbuf = (g_a, g_b)
        dbuf = (d_a, d_b)
        msg = (m_a, m_b)
        semg = (sg_a, sg_b)
        semd = (sdm_a, sdm_b)
        sems = (ss_a, ss_b)

        pltpu.sync_copy(z_hbm.at[pl.ds(row0, ROWS_PER_S)],
                        acc_sh.at[pl.ds(row0, ROWS_PER_S)])
        plsc.subcore_barrier()

        def fetch(j, b):
            pltpu.sync_copy(idx_hbm.at[pl.ds(wid * NCHUNK + j, 1)], idxv[b])
            pltpu.async_copy(g_hbm.at[idxv[b].at[0, pl.ds(0, CHUNK)]],
                             gbuf[b], semg[b])
            pltpu.async_copy(d_hbm.at[idxv[b].at[0, pl.ds(CHUNK, CHUNK)]],
                             dbuf[b], semd[b])

        def wait_gathers(b):
            pltpu.make_async_copy(g_hbm.at[idxv[b].at[0, pl.ds(0, CHUNK)]],
                                  gbuf[b], semg[b]).wait()
            pltpu.make_async_copy(d_hbm.at[idxv[b].at[0, pl.ds(CHUNK, CHUNK)]],
                                  dbuf[b], semd[b]).wait()

        def compute(b):
            for i in range(CHUNK // 16):
                sdst[b][pl.ds(16 * i, 16)] = idxv[b][0, pl.ds(CHUNK + 16 * i, 16)]
            compute_chunk(gbuf[b], dbuf[b], msg[b])

        def issue_scatter(b):
            pltpu.async_copy(msg[b], acc_sh.at[sdst[b]], sems[b], add=True)

        def wait_scatter(b):
            pltpu.make_async_copy(msg[b], acc_sh.at[sdst[b]], sems[b]).wait()

        fetch(0, 0)
        fetch(1, 1)
        for b in (0, 1):
            wait_gathers(b)
            compute(b)
            issue_scatter(b)
            fetch(b + 2, b)

        def pair(t, carry):
            for b in (0, 1):
                j = 2 * t + b
                wait_scatter(b)
                wait_gathers(b)
                compute(b)
                issue_scatter(b)

                @pl.when(j + 2 < NCHUNK)
                def _():
                    fetch(j + 2, b)
            return carry

        lax.fori_loop(1, (NCHUNK - 1) // 2, pair, 0)

        wait_scatter(0)
        wait_gathers(0)
        compute(0)
        issue_scatter(0)

        wait_scatter(1)
        wait_scatter(0)
        plsc.subcore_barrier()
        pltpu.sync_copy(acc_sh.at[pl.ds(row0, ROWS_PER_S)],
                        out_hbm.at[c, pl.ds(row0, ROWS_PER_S)])

    return body


def _sc_edge1(idx_hbm, g_hbm, d_hbm, z_hbm, zd_hbm, out_hbm, outd_hbm,
              acc_sh, den,
              idx_a, idx_b, sd_a, sd_b, sp_a, sp_b,
              g_a, g_b, d_a, d_b, m_a, m_b,
              sg_a, sg_b, sdm_a, sdm_b, ss_a, ss_b):
    c = lax.axis_index("c")
    s = lax.axis_index("s")
    wid = s * NC + c
    row0 = s * ROWS_PER_S

    idxv = (idx_a, idx_b)
    sdst = (sd_a, sd_b)
    sdp = (sp_a, sp_b)
    gbuf = (g_a, g_b)
    dbuf = (d_a, d_b)
    msg = (m_a, m_b)
    semg = (sg_a, sg_b)
    semd = (sdm_a, sdm_b)
    sems = (ss_a, ss_b)

    pltpu.sync_copy(z_hbm.at[pl.ds(row0, ROWS_PER_S)],
                    acc_sh.at[pl.ds(row0, ROWS_PER_S)])
    pltpu.sync_copy(zd_hbm, den)
    plsc.subcore_barrier()

    iota = lax.broadcasted_iota(jnp.int32, (16,), 0)
    idx_m = (iota & 7) + 8
    idx_w = iota >> 1
    col8 = iota & 7
    lt8 = iota < 8

    def fetch(j, b):
        pltpu.sync_copy(idx_hbm.at[pl.ds(wid * NCHUNK + j, 1)], idxv[b])
        pltpu.async_copy(g_hbm.at[idxv[b].at[0, pl.ds(0, CHUNK)]],
                         gbuf[b], semg[b])
        pltpu.async_copy(d_hbm.at[idxv[b].at[0, pl.ds(CHUNK, CHUNK)]],
                         dbuf[b], semd[b])

    def wait_gathers(b):
        pltpu.make_async_copy(g_hbm.at[idxv[b].at[0, pl.ds(0, CHUNK)]],
                              gbuf[b], semg[b]).wait()
        pltpu.make_async_copy(d_hbm.at[idxv[b].at[0, pl.ds(CHUNK, CHUNK)]],
                              dbuf[b], semd[b]).wait()

    def compute(b):
        for i in range(CHUNK // 16):
            v = idxv[b][0, pl.ds(CHUNK + 16 * i, 16)]
            sdst[b][pl.ds(16 * i, 16)] = v
            sdp[b][pl.ds(16 * i, 16)] = v

        def edge(e, carry):
            g0 = gbuf[b][e, pl.ds(0, 16)]
            d0 = dbuf[b][e, pl.ds(0, 16)]
            s0 = g0 + d0
            lk = _leaky(s0)
            mv = _perm(s0, idx_m)
            ex = jnp.exp(lk - mv)
            wexp = _perm(ex, idx_w)
            msg[b][e, pl.ds(0, 16)] = wexp * gbuf[b][e, pl.ds(16, 16)]
            msg[b][e, pl.ds(16, 16)] = wexp * gbuf[b][e, pl.ds(32, 16)]
            dvv = sdp[b][pl.ds(e, 16)]
            rowi = _perm(dvv, iota * 0)
            plsc.addupdate_scatter(den, [rowi, col8], ex, mask=lt8)
            return carry

        lax.fori_loop(0, CHUNK, edge, 0, unroll=4)

    def issue_scatter(b):
        pltpu.async_copy(msg[b], acc_sh.at[sdst[b]], sems[b], add=True)

    def wait_scatter(b):
        pltpu.make_async_copy(msg[b], acc_sh.at[sdst[b]], sems[b]).wait()

    fetch(0, 0)
    fetch(1, 1)
    for b in (0, 1):
        wait_gathers(b)
        compute(b)
        issue_scatter(b)
        fetch(b + 2, b)

    def pair(t, carry):
        for b in (0, 1):
            j = 2 * t + b
            wait_scatter(b)
            wait_gathers(b)
            compute(b)
            issue_scatter(b)

            @pl.when(j + 2 < NCHUNK)
            def _():
                fetch(j + 2, b)
        return carry

    lax.fori_loop(1, (NCHUNK - 1) // 2, pair, 0)

    wait_scatter(0)
    wait_gathers(0)
    compute(0)
    issue_scatter(0)

    wait_scatter(1)
    wait_scatter(0)
    plsc.subcore_barrier()
    pltpu.sync_copy(acc_sh.at[pl.ds(row0, ROWS_PER_S)],
                    out_hbm.at[c, pl.ds(row0, ROWS_PER_S)])
    pltpu.sync_copy(den, outd_hbm.at[c, s])


def _compute_chunk2(gbuf, dbuf, msg):
    iota = lax.broadcasted_iota(jnp.int32, (16,), 0)
    half = iota >> 3
    coli = iota & 7
    idx_m = half * 8 + 1
    idx_w = half * 8
    lt3 = coli < 3
    idx_sh = jnp.where(lt3, half * 8 + coli + 1, half * 8)

    def pair(e2, carry):
        rowi = half + 2 * e2
        g2 = plsc.load_gather(gbuf, [rowi, coli])
        d2 = plsc.load_gather(dbuf, [rowi, coli])
        s0 = g2 + d2
        lk = _leaky(s0)
        mv = _perm(d2, idx_m)
        ex = jnp.exp(lk - mv)
        wv = _perm(ex, idx_w)
        gs = _perm(g2, idx_sh)
        plsc.store_scatter(msg, [rowi, coli], jnp.where(lt3, wv * gs, 0.0))
        return carry

    lax.fori_loop(0, CHUNK // 2, pair, 0, unroll=4)


_sc_edge2 = _sc_edge_pipeline(_compute_chunk2)


_SC_PARAMS = pltpu.CompilerParams(use_tc_tiling_on_sc=False,
                                  needs_layout_passes=False)

_sc1 = functools.partial(
    pl.kernel, _sc_edge1,
    out_type=[jax.ShapeDtypeStruct((NC, N, 32), jnp.float32),
              jax.ShapeDtypeStruct((NC, NS, N, 8), jnp.float32)],
    mesh=_MESH,
    compiler_params=_SC_PARAMS,
    scratch_types=(
        [pltpu.VMEM_SHARED((N, 32), jnp.float32),
         pltpu.VMEM((N, 8), jnp.float32)]
        + [pltpu.VMEM((1, 2 * CHUNK), jnp.int32)] * 2
        + [pltpu.VMEM((CHUNK,), jnp.int32)] * 2
        + [pltpu.VMEM((CHUNK + 16,), jnp.int32)] * 2
        + [pltpu.VMEM((CHUNK, 48), jnp.float32)] * 2
        + [pltpu.VMEM((CHUNK, 16), jnp.float32)] * 2
        + [pltpu.VMEM((CHUNK, 32), jnp.float32)] * 2
        + [pltpu.SemaphoreType.DMA] * 6
    ),
)()

_sc2 = functools.partial(
    pl.kernel, _sc_edge2,
    out_type=jax.ShapeDtypeStruct((NC, N, 8), jnp.float32),
    mesh=_MESH,
    compiler_params=_SC_PARAMS,
    scratch_types=(
        [pltpu.VMEM_SHARED((N, 8), jnp.float32)]
        + [pltpu.VMEM((1, 2 * CHUNK), jnp.int32)] * 2
        + [pltpu.VMEM((CHUNK,), jnp.int32)] * 2
        + [pltpu.VMEM((CHUNK, 8), jnp.float32)] * 6
        + [pltpu.SemaphoreType.DMA] * 6
    ),
)()


@jax.jit
def kernel(x, edge_index, W1_src, W1_dst, att1_src, att1_dst, b1,
           W2_src, W2_dst, att2_src, att2_dst, b2):
    src = edge_index[0]
    dst = edge_index[1]
    idx_rows = jnp.concatenate([src.reshape(-1, CHUNK), dst.reshape(-1, CHUNK)],
                               axis=1)

    W1s3 = W1_src.reshape(F_IN, H1, HID)
    W1d3 = W1_dst.reshape(F_IN, H1, HID)
    Vs = jnp.einsum('fhc,hc->fh', W1s3, att1_src)
    Vd = jnp.einsum('fhc,hc->fh', W1d3, att1_dst)
    Us = jnp.einsum('fhc,hco->fho', W1s3, W2_src.reshape(H1, HID, OUT))
    Ud = jnp.einsum('fhc,hco->fho', W1s3, W2_dst.reshape(H1, HID, OUT))
    Bg = jnp.concatenate([Vs, jnp.zeros((F_IN, 8), jnp.float32),
                          Us.reshape(F_IN, 16), Ud.reshape(F_IN, 16)], axis=1)
    cs = (b1 @ W2_src).reshape(1, OUT)
    cd = (b1 @ W2_dst).reshape(1, OUT)

    gsrc, dtab = pl.pallas_call(
        _tc1_body,
        out_shape=[jax.ShapeDtypeStruct((N, 48), jnp.float32),
                   jax.ShapeDtypeStruct((N, 16), jnp.float32)],
    )(x, Bg, Vd)

    z32 = jnp.zeros((N, 32), jnp.float32)
    z8 = jnp.zeros((N, 8), jnp.float32)
    p1, pd1 = _sc1(idx_rows, gsrc, dtab, z32, z8)

    inv625 = pl.pallas_call(
        _tc2a_body,
        out_shape=jax.ShapeDtypeStruct((N // 16, 128), jnp.float32),
    )(pd1.reshape(NW, N // 16, 128))
    inv8 = inv625.reshape(N, 8)
    g2, d2 = pl.pallas_call(
        _tc2_body,
        out_shape=[jax.ShapeDtypeStruct((N, 8), jnp.float32),
                   jax.ShapeDtypeStruct((N, 8), jnp.float32)],
    )(p1, inv8, cs, cd,
      att2_src.reshape(OUT, 1), att2_dst.reshape(OUT, 1))

    p2 = _sc2(idx_rows, g2, d2, z8)

    out = pl.pallas_call(
        _tc3_body,
        out_shape=jax.ShapeDtypeStruct((N, OUT), jnp.float32),
    )(p2, b2.reshape(1, OUT))
    return out


# speedup vs baseline: 1.1121x; 1.1121x over previous
import functools

import jax
import jax.numpy as jnp
from jax import lax
from jax.experimental import pallas as pl
from jax.experimental.pallas import tpu as pltpu
from jax.experimental.pallas import tpu_sc as plsc

N = 10000
E = 320000
F_IN = 128
HID = 512
H1 = 8
OUT = 2

NC = 2
NS = 16
NW = NC * NS
EDGES_PER_W = E // NW
CHUNK = 80
NCHUNK = EDGES_PER_W // CHUNK
ROWS_PER_S = N // NS

_LEAK = 0.2


def _leaky(v):
    return jnp.where(v > 0, v, _LEAK * v)


def _perm(v, idx):
    dnums = lax.GatherDimensionNumbers(
        offset_dims=(), collapsed_slice_dims=(0,), start_index_map=(0,))
    return lax.gather(v, idx[:, None], dnums, (1,),
                      mode=lax.GatherScatterMode.PROMISE_IN_BOUNDS)


def _tc1_body(x_ref, bg_ref, bd_ref, gs_ref, d_ref):
    xv = x_ref[...]
    gs = jnp.dot(xv, bg_ref[...], preferred_element_type=jnp.float32)
    gs_ref[...] = gs
    a1d = jnp.dot(xv, bd_ref[...], preferred_element_type=jnp.float32)
    a1s = gs[:, 0:8]
    gmax = jnp.max(a1s, axis=0, keepdims=True)
    m = gmax + a1d
    d_ref[...] = jnp.concatenate([a1d, _leaky(m)], axis=1)


def _tc2_body(p_ref, cs_ref, cd_ref, a2s_w_ref, a2d_w_ref,
              g2_ref, d2_ref):
    acc = p_ref[0] + p_ref[1]
    inv16 = 1.0 / (acc[:, 0:16] + 1e-16)
    zsi = acc[:, 16:32] * inv16
    zdi = acc[:, 32:48] * inv16
    lane = lax.broadcasted_iota(jnp.int32, (16, 2), 0)
    col = lax.broadcasted_iota(jnp.int32, (16, 2), 1)
    sel = (lane % 2 == col).astype(jnp.float32)
    xs2 = jnp.dot(zsi, sel, preferred_element_type=jnp.float32) + cs_ref[...]
    xd2 = jnp.dot(zdi, sel, preferred_element_type=jnp.float32) + cd_ref[...]
    a2s = jnp.dot(xs2, a2s_w_ref[...], preferred_element_type=jnp.float32)
    a2d = jnp.dot(xd2, a2d_w_ref[...], preferred_element_type=jnp.float32)
    m2 = _leaky(jnp.max(a2s, axis=0, keepdims=True) + a2d)
    ones = jnp.ones_like(a2s)
    zeros4 = jnp.zeros((xs2.shape[0], 4), jnp.float32)
    g2_ref[...] = jnp.concatenate([a2s, ones, xs2, zeros4], axis=1)
    d2_ref[...] = jnp.concatenate([a2d, m2, zeros4, jnp.zeros_like(xs2)], axis=1)


def _tc3_body(p2_ref, b2_ref, out_ref):
    acc2 = p2_ref[0] + p2_ref[1]
    denom = acc2[:, 0:1]
    num = acc2[:, 1:3]
    out_ref[...] = num / (denom + 1e-16) + b2_ref[...]


_MESH = plsc.VectorSubcoreMesh(core_axis_name="c", subcore_axis_name="s",
                               num_cores=NC, num_subcores=NS)


def _sc_edge_pipeline(compute_chunk):
    def body(idx_hbm, g_hbm, d_hbm, z_hbm, out_hbm,
             acc_sh,
             idx_a, idx_b, sd_a, sd_b,
             g_a, g_b, d_a, d_b, m_a, m_b,
             sg_a, sg_b, sdm_a, sdm_b, ss_a, ss_b):
        c = lax.axis_index("c")
        s = lax.axis_index("s")
        wid = s * NC + c
        row0 = s * ROWS_PER_S

        idxv = (idx_a, idx_b)
        sdst = (sd_a, sd_b)
        gbuf = (g_a, g_b)
        dbuf = (d_a, d_b)
        msg = (m_a, m_b)
        semg = (sg_a, sg_b)
        semd = (sdm_a, sdm_b)
        sems = (ss_a, ss_b)

        pltpu.sync_copy(z_hbm.at[pl.ds(row0, ROWS_PER_S)],
                        acc_sh.at[pl.ds(row0, ROWS_PER_S)])
        plsc.subcore_barrier()

        def fetch(j, b):
            pltpu.sync_copy(idx_hbm.at[pl.ds(wid * NCHUNK + j, 1)], idxv[b])
            pltpu.async_copy(g_hbm.at[idxv[b].at[0, pl.ds(0, CHUNK)]],
                             gbuf[b], semg[b])
            pltpu.async_copy(d_hbm.at[idxv[b].at[0, pl.ds(CHUNK, CHUNK)]],
                             dbuf[b], semd[b])

        def wait_gathers(b):
            pltpu.make_async_copy(g_hbm.at[idxv[b].at[0, pl.ds(0, CHUNK)]],
                                  gbuf[b], semg[b]).wait()
            pltpu.make_async_copy(d_hbm.at[idxv[b].at[0, pl.ds(CHUNK, CHUNK)]],
                                  dbuf[b], semd[b]).wait()

        def compute(b):
            for i in range(CHUNK // 16):
                sdst[b][pl.ds(16 * i, 16)] = idxv[b][0, pl.ds(CHUNK + 16 * i, 16)]
            compute_chunk(gbuf[b], dbuf[b], msg[b])

        def issue_scatter(b):
            pltpu.async_copy(msg[b], acc_sh.at[sdst[b]], sems[b], add=True)

        def wait_scatter(b):
            pltpu.make_async_copy(msg[b], acc_sh.at[sdst[b]], sems[b]).wait()

        fetch(0, 0)
        fetch(1, 1)
        for b in (0, 1):
            wait_gathers(b)
            compute(b)
            issue_scatter(b)
            fetch(b + 2, b)

        def pair(t, carry):
            for b in (0, 1):
                j = 2 * t + b
                wait_scatter(b)
                wait_gathers(b)
                compute(b)
                issue_scatter(b)

                @pl.when(j + 2 < NCHUNK)
                def _():
                    fetch(j + 2, b)
            return carry

        lax.fori_loop(1, (NCHUNK - 1) // 2, pair, 0)

        wait_scatter(0)
        wait_gathers(0)
        compute(0)
        issue_scatter(0)

        wait_scatter(1)
        wait_scatter(0)
        plsc.subcore_barrier()
        pltpu.sync_copy(acc_sh.at[pl.ds(row0, ROWS_PER_S)],
                        out_hbm.at[c, pl.ds(row0, ROWS_PER_S)])

    return body


def _compute_chunk1(gbuf, dbuf, msg):
    iota = lax.broadcasted_iota(jnp.int32, (16,), 0)
    idx_m = (iota & 7) + 8
    idx_w = iota >> 1

    def edge(e, carry):
        g0 = gbuf[e, pl.ds(0, 16)]
        d0 = dbuf[e, pl.ds(0, 16)]
        s0 = g0 + d0
        lk = _leaky(s0)
        mv = _perm(s0, idx_m)
        ex = jnp.exp(lk - mv)
        wexp = _perm(ex, idx_w)
        msg[e, pl.ds(0, 16)] = wexp
        msg[e, pl.ds(16, 16)] = wexp * gbuf[e, pl.ds(16, 16)]
        msg[e, pl.ds(32, 16)] = wexp * gbuf[e, pl.ds(32, 16)]
        return carry

    lax.fori_loop(0, CHUNK, edge, 0, unroll=4)


_sc_edge1 = _sc_edge_pipeline(_compute_chunk1)


def _compute_chunk2(gbuf, dbuf, msg):
    iota = lax.broadcasted_iota(jnp.int32, (16,), 0)
    half = iota >> 3
    coli = iota & 7
    idx_m = half * 8 + 1
    idx_w = half * 8
    lt3 = coli < 3
    idx_sh = jnp.where(lt3, half * 8 + coli + 1, half * 8)

    def pair(e2, carry):
        rowi = half + 2 * e2
        g2 = plsc.load_gather(gbuf, [rowi, coli])
        d2 = plsc.load_gather(dbuf, [rowi, coli])
        s0 = g2 + d2
        lk = _leaky(s0)
        mv = _perm(d2, idx_m)
        ex = jnp.exp(lk - mv)
        wv = _perm(ex, idx_w)
        gs = _perm(g2, idx_sh)
        plsc.store_scatter(msg, [rowi, coli], jnp.where(lt3, wv * gs, 0.0))
        return carry

    lax.fori_loop(0, CHUNK // 2, pair, 0, unroll=4)


_sc_edge2 = _sc_edge_pipeline(_compute_chunk2)


_SC_PARAMS = pltpu.CompilerParams(use_tc_tiling_on_sc=False,
                                  needs_layout_passes=False)

_sc1 = functools.partial(
    pl.kernel, _sc_edge1,
    out_type=jax.ShapeDtypeStruct((NC, N, 48), jnp.float32),
    mesh=_MESH,
    compiler_params=_SC_PARAMS,
    scratch_types=(
        [pltpu.VMEM_SHARED((N, 48), jnp.float32)]
        + [pltpu.VMEM((1, 2 * CHUNK), jnp.int32)] * 2
        + [pltpu.VMEM((CHUNK,), jnp.int32)] * 2
        + [pltpu.VMEM((CHUNK, 48), jnp.float32)] * 2
        + [pltpu.VMEM((CHUNK, 16), jnp.float32)] * 2
        + [pltpu.VMEM((CHUNK, 48), jnp.float32)] * 2
        + [pltpu.SemaphoreType.DMA] * 6
    ),
)()

_sc2 = functools.partial(
    pl.kernel, _sc_edge2,
    out_type=jax.ShapeDtypeStruct((NC, N, 8), jnp.float32),
    mesh=_MESH,
    compiler_params=_SC_PARAMS,
    scratch_types=(
        [pltpu.VMEM_SHARED((N, 8), jnp.float32)]
        + [pltpu.VMEM((1, 2 * CHUNK), jnp.int32)] * 2
        + [pltpu.VMEM((CHUNK,), jnp.int32)] * 2
        + [pltpu.VMEM((CHUNK, 8), jnp.float32)] * 6
        + [pltpu.SemaphoreType.DMA] * 6
    ),
)()


@jax.jit
def kernel(x, edge_index, W1_src, W1_dst, att1_src, att1_dst, b1,
           W2_src, W2_dst, att2_src, att2_dst, b2):
    src = edge_index[0]
    dst = edge_index[1]
    idx_rows = jnp.concatenate([src.reshape(-1, CHUNK), dst.reshape(-1, CHUNK)],
                               axis=1)

    W1s3 = W1_src.reshape(F_IN, H1, HID)
    W1d3 = W1_dst.reshape(F_IN, H1, HID)
    Vs = jnp.einsum('fhc,hc->fh', W1s3, att1_src)
    Vd = jnp.einsum('fhc,hc->fh', W1d3, att1_dst)
    Us = jnp.einsum('fhc,hco->fho', W1s3, W2_src.reshape(H1, HID, OUT))
    Ud = jnp.einsum('fhc,hco->fho', W1s3, W2_dst.reshape(H1, HID, OUT))
    Bg = jnp.concatenate([Vs, jnp.zeros((F_IN, 8), jnp.float32),
                          Us.reshape(F_IN, 16), Ud.reshape(F_IN, 16)], axis=1)
    cs = (b1 @ W2_src).reshape(1, OUT)
    cd = (b1 @ W2_dst).reshape(1, OUT)

    gsrc, dtab = pl.pallas_call(
        _tc1_body,
        out_shape=[jax.ShapeDtypeStruct((N, 48), jnp.float32),
                   jax.ShapeDtypeStruct((N, 16), jnp.float32)],
    )(x, Bg, Vd)

    z48 = jnp.zeros((N, 48), jnp.float32)
    z8 = jnp.zeros((N, 8), jnp.float32)
    p1 = _sc1(idx_rows, gsrc, dtab, z48)

    g2, d2 = pl.pallas_call(
        _tc2_body,
        out_shape=[jax.ShapeDtypeStruct((N, 8), jnp.float32),
                   jax.ShapeDtypeStruct((N, 8), jnp.float32)],
    )(p1, cs, cd,
      att2_src.reshape(OUT, 1), att2_dst.reshape(OUT, 1))

    p2 = _sc2(idx_rows, g2, d2, z8)

    out = pl.pallas_call(
        _tc3_body,
        out_shape=jax.ShapeDtypeStruct((N, OUT), jnp.float32),
    )(p2, b2.reshape(1, OUT))
    return out


# speedup vs baseline: 1.1125x; 1.0004x over previous
import functools

import jax
import jax.numpy as jnp
from jax import lax
from jax.experimental import pallas as pl
from jax.experimental.pallas import tpu as pltpu
from jax.experimental.pallas import tpu_sc as plsc

N = 10000
E = 320000
F_IN = 128
HID = 512
H1 = 8
OUT = 2

NC = 2
NS = 16
NW = NC * NS
EDGES_PER_W = E // NW
CHUNK = 80
NCHUNK = EDGES_PER_W // CHUNK
ROWS_PER_S = N // NS

_LEAK = 0.2


def _leaky(v):
    return jnp.where(v > 0, v, _LEAK * v)


def _perm(v, idx):
    dnums = lax.GatherDimensionNumbers(
        offset_dims=(), collapsed_slice_dims=(0,), start_index_map=(0,))
    return lax.gather(v, idx[:, None], dnums, (1,),
                      mode=lax.GatherScatterMode.PROMISE_IN_BOUNDS)


def _tc1_body(x_ref, bg_ref, bd_ref, gs_ref, d_ref):
    xv = x_ref[...]
    gs = jnp.dot(xv, bg_ref[...], preferred_element_type=jnp.float32)
    gs_ref[...] = gs
    a1d = jnp.dot(xv, bd_ref[...], preferred_element_type=jnp.float32)
    a1s = gs[:, 0:8]
    gmax = jnp.max(a1s, axis=0, keepdims=True)
    m = gmax + a1d
    d_ref[...] = jnp.concatenate([a1d, _leaky(m)], axis=1)


def _tc2_body(p_ref, cs_ref, cd_ref, a2s_w_ref, a2d_w_ref,
              g2_ref, d2_ref):
    acc = p_ref[0] + p_ref[1]
    inv16 = 1.0 / (acc[:, 0:16] + 1e-16)
    zsi = acc[:, 16:32] * inv16
    zdi = acc[:, 32:48] * inv16
    lane = lax.broadcasted_iota(jnp.int32, (16, 2), 0)
    col = lax.broadcasted_iota(jnp.int32, (16, 2), 1)
    sel = (lane % 2 == col).astype(jnp.float32)
    xs2 = jnp.dot(zsi, sel, preferred_element_type=jnp.float32) + cs_ref[...]
    xd2 = jnp.dot(zdi, sel, preferred_element_type=jnp.float32) + cd_ref[...]
    a2s = jnp.dot(xs2, a2s_w_ref[...], preferred_element_type=jnp.float32)
    a2d = jnp.dot(xd2, a2d_w_ref[...], preferred_element_type=jnp.float32)
    m2 = _leaky(jnp.max(a2s, axis=0, keepdims=True) + a2d)
    ones = jnp.ones_like(a2s)
    zeros4 = jnp.zeros((xs2.shape[0], 4), jnp.float32)
    g2_ref[...] = jnp.concatenate([a2s, ones, xs2, zeros4], axis=1)
    d2_ref[...] = jnp.concatenate([a2d, m2, zeros4, jnp.zeros_like(xs2)], axis=1)


def _tc3_body(p2_ref, b2_ref, out_ref):
    acc2 = p2_ref[0] + p2_ref[1]
    denom = acc2[:, 0:1]
    num = acc2[:, 1:3]
    out_ref[...] = num / (denom + 1e-16) + b2_ref[...]


_MESH = plsc.VectorSubcoreMesh(core_axis_name="c", subcore_axis_name="s",
                               num_cores=NC, num_subcores=NS)


def _sc_edge_pipeline(compute_chunk):
    def body(idx_hbm, g_hbm, d_hbm, z_hbm, out_hbm,
             acc_sh,
             idx_a, idx_b, sd_a, sd_b,
             g_a, g_b, d_a, d_b, m_a, m_b,
             sg_a, sg_b, sdm_a, sdm_b, ss_a, ss_b):
        c = lax.axis_index("c")
        s = lax.axis_index("s")
        wid = s * NC + c
        row0 = s * ROWS_PER_S

        idxv = (idx_a, idx_b)
        sdst = (sd_a, sd_b)
        gbuf = (g_a, g_b)
        dbuf = (d_a, d_b)
        msg = (m_a, m_b)
        semg = (sg_a, sg_b)
        semd = (sdm_a, sdm_b)
        sems = (ss_a, ss_b)

        pltpu.sync_copy(z_hbm.at[pl.ds(row0, ROWS_PER_S)],
                        acc_sh.at[pl.ds(row0, ROWS_PER_S)])
        plsc.subcore_barrier()

        def fetch(j, b):
            pltpu.sync_copy(idx_hbm.at[pl.ds(wid * NCHUNK + j, 1)], idxv[b])
            pltpu.async_copy(g_hbm.at[idxv[b].at[0, pl.ds(0, CHUNK)]],
                             gbuf[b], semg[b])
            pltpu.async_copy(d_hbm.at[idxv[b].at[0, pl.ds(CHUNK, CHUNK)]],
                             dbuf[b], semd[b])

        def wait_gathers(b):
            pltpu.make_async_copy(g_hbm.at[idxv[b].at[0, pl.ds(0, CHUNK)]],
                                  gbuf[b], semg[b]).wait()
            pltpu.make_async_copy(d_hbm.at[idxv[b].at[0, pl.ds(CHUNK, CHUNK)]],
                                  dbuf[b], semd[b]).wait()

        def compute(b):
            for i in range(CHUNK // 16):
                sdst[b][pl.ds(16 * i, 16)] = idxv[b][0, pl.ds(CHUNK + 16 * i, 16)]
            compute_chunk(gbuf[b], dbuf[b], msg[b])

        def issue_scatter(b):
            pltpu.async_copy(msg[b], acc_sh.at[sdst[b]], sems[b], add=True)

        def wait_scatter(b):
            pltpu.make_async_copy(msg[b], acc_sh.at[sdst[b]], sems[b]).wait()

        fetch(0, 0)
        fetch(1, 1)
        for b in (0, 1):
            wait_gathers(b)
            compute(b)
            issue_scatter(b)
            fetch(b + 2, b)

        def pair(t, carry):
            for b in (0, 1):
                j = 2 * t + b
                wait_scatter(b)
                wait_gathers(b)
                compute(b)
                issue_scatter(b)

                @pl.when(j + 2 < NCHUNK)
                def _():
                    fetch(j + 2, b)
            return carry

        lax.fori_loop(1, (NCHUNK - 1) // 2, pair, 0)

        wait_scatter(0)
        wait_gathers(0)
        compute(0)
        issue_scatter(0)

        wait_scatter(1)
        wait_scatter(0)
        plsc.subcore_barrier()
        pltpu.sync_copy(acc_sh.at[pl.ds(row0, ROWS_PER_S)],
                        out_hbm.at[c, pl.ds(row0, ROWS_PER_S)])

    return body


def _compute_chunk1(gbuf, dbuf, msg):
    iota = lax.broadcasted_iota(jnp.int32, (16,), 0)
    idx_m = (iota & 7) + 8
    idx_w = iota >> 1

    def edge(e, carry):
        g0 = gbuf[e, pl.ds(0, 16)]
        d0 = dbuf[e, pl.ds(0, 16)]
        s0 = g0 + d0
        lk = _leaky(s0)
        mv = _perm(s0, idx_m)
        ex = jnp.exp(lk - mv)
        wexp = _perm(ex, idx_w)
        msg[e, pl.ds(0, 16)] = wexp
        msg[e, pl.ds(16, 16)] = wexp * gbuf[e, pl.ds(16, 16)]
        msg[e, pl.ds(32, 16)] = wexp * gbuf[e, pl.ds(32, 16)]
        return carry

    lax.fori_loop(0, CHUNK, edge, 0, unroll=8)


_sc_edge1 = _sc_edge_pipeline(_compute_chunk1)


def _compute_chunk2(gbuf, dbuf, msg):
    iota = lax.broadcasted_iota(jnp.int32, (16,), 0)
    half = iota >> 3
    coli = iota & 7
    idx_m = half * 8 + 1
    idx_w = half * 8
    lt3 = coli < 3
    idx_sh = jnp.where(lt3, half * 8 + coli + 1, half * 8)

    def pair(e2, carry):
        rowi = half + 2 * e2
        g2 = plsc.load_gather(gbuf, [rowi, coli])
        d2 = plsc.load_gather(dbuf, [rowi, coli])
        s0 = g2 + d2
        lk = _leaky(s0)
        mv = _perm(d2, idx_m)
        ex = jnp.exp(lk - mv)
        wv = _perm(ex, idx_w)
        gs = _perm(g2, idx_sh)
        plsc.store_scatter(msg, [rowi, coli], jnp.where(lt3, wv * gs, 0.0))
        return carry

    lax.fori_loop(0, CHUNK // 2, pair, 0, unroll=8)


_sc_edge2 = _sc_edge_pipeline(_compute_chunk2)


_SC_PARAMS = pltpu.CompilerParams(use_tc_tiling_on_sc=False,
                                  needs_layout_passes=False)

_sc1 = functools.partial(
    pl.kernel, _sc_edge1,
    out_type=jax.ShapeDtypeStruct((NC, N, 48), jnp.float32),
    mesh=_MESH,
    compiler_params=_SC_PARAMS,
    scratch_types=(
        [pltpu.VMEM_SHARED((N, 48), jnp.float32)]
        + [pltpu.VMEM((1, 2 * CHUNK), jnp.int32)] * 2
        + [pltpu.VMEM((CHUNK,), jnp.int32)] * 2
        + [pltpu.VMEM((CHUNK, 48), jnp.float32)] * 2
        + [pltpu.VMEM((CHUNK, 16), jnp.float32)] * 2
        + [pltpu.VMEM((CHUNK, 48), jnp.float32)] * 2
        + [pltpu.SemaphoreType.DMA] * 6
    ),
)()

_sc2 = functools.partial(
    pl.kernel, _sc_edge2,
    out_type=jax.ShapeDtypeStruct((NC, N, 8), jnp.float32),
    mesh=_MESH,
    compiler_params=_SC_PARAMS,
    scratch_types=(
        [pltpu.VMEM_SHARED((N, 8), jnp.float32)]
        + [pltpu.VMEM((1, 2 * CHUNK), jnp.int32)] * 2
        + [pltpu.VMEM((CHUNK,), jnp.int32)] * 2
        + [pltpu.VMEM((CHUNK, 8), jnp.float32)] * 6
        + [pltpu.SemaphoreType.DMA] * 6
    ),
)()


@jax.jit
def kernel(x, edge_index, W1_src, W1_dst, att1_src, att1_dst, b1,
           W2_src, W2_dst, att2_src, att2_dst, b2):
    src = edge_index[0]
    dst = edge_index[1]
    idx_rows = jnp.concatenate([src.reshape(-1, CHUNK), dst.reshape(-1, CHUNK)],
                               axis=1)

    W1s3 = W1_src.reshape(F_IN, H1, HID)
    W1d3 = W1_dst.reshape(F_IN, H1, HID)
    Vs = jnp.einsum('fhc,hc->fh', W1s3, att1_src)
    Vd = jnp.einsum('fhc,hc->fh', W1d3, att1_dst)
    Us = jnp.einsum('fhc,hco->fho', W1s3, W2_src.reshape(H1, HID, OUT))
    Ud = jnp.einsum('fhc,hco->fho', W1s3, W2_dst.reshape(H1, HID, OUT))
    Bg = jnp.concatenate([Vs, jnp.zeros((F_IN, 8), jnp.float32),
                          Us.reshape(F_IN, 16), Ud.reshape(F_IN, 16)], axis=1)
    cs = (b1 @ W2_src).reshape(1, OUT)
    cd = (b1 @ W2_dst).reshape(1, OUT)

    gsrc, dtab = pl.pallas_call(
        _tc1_body,
        out_shape=[jax.ShapeDtypeStruct((N, 48), jnp.float32),
                   jax.ShapeDtypeStruct((N, 16), jnp.float32)],
    )(x, Bg, Vd)

    z48 = jnp.zeros((N, 48), jnp.float32)
    z8 = jnp.zeros((N, 8), jnp.float32)
    p1 = _sc1(idx_rows, gsrc, dtab, z48)

    g2, d2 = pl.pallas_call(
        _tc2_body,
        out_shape=[jax.ShapeDtypeStruct((N, 8), jnp.float32),
                   jax.ShapeDtypeStruct((N, 8), jnp.float32)],
    )(p1, cs, cd,
      att2_src.reshape(OUT, 1), att2_dst.reshape(OUT, 1))

    p2 = _sc2(idx_rows, g2, d2, z8)

    out = pl.pallas_call(
        _tc3_body,
        out_shape=jax.ShapeDtypeStruct((N, OUT), jnp.float32),
    )(p2, b2.reshape(1, OUT))
    return out


# speedup vs baseline: 1.1488x; 1.0326x over previous
import functools

import jax
import jax.numpy as jnp
from jax import lax
from jax.experimental import pallas as pl
from jax.experimental.pallas import tpu as pltpu
from jax.experimental.pallas import tpu_sc as plsc

N = 10000
E = 320000
F_IN = 128
HID = 512
H1 = 8
OUT = 2

NC = 2
NS = 16
NW = NC * NS
EDGES_PER_W = E // NW
CHUNK = 80
NCHUNK = EDGES_PER_W // CHUNK
ROWS_PER_S = N // NS

_LEAK = 0.2


def _leaky(v):
    return jnp.where(v > 0, v, _LEAK * v)


def _perm(v, idx):
    dnums = lax.GatherDimensionNumbers(
        offset_dims=(), collapsed_slice_dims=(0,), start_index_map=(0,))
    return lax.gather(v, idx[:, None], dnums, (1,),
                      mode=lax.GatherScatterMode.PROMISE_IN_BOUNDS)


def _tc1_body(x_ref, bg_ref, bd_ref, gs_ref, d_ref):
    xv = x_ref[...]
    gs = jnp.dot(xv, bg_ref[...], preferred_element_type=jnp.float32)
    gs_ref[...] = gs
    a1d = jnp.dot(xv, bd_ref[...], preferred_element_type=jnp.float32)
    a1s = gs[:, 0:8]
    gmax = jnp.max(a1s, axis=0, keepdims=True)
    m = gmax + a1d
    d_ref[...] = jnp.concatenate([a1d, _leaky(m)], axis=1)


def _tc2_body(p_ref, pw_ref, cs_ref, cd_ref, a2s_w_ref, a2d_w_ref,
              g2_ref, d2_ref):
    acc = p_ref[0] + p_ref[1]
    inv = 1.0 / (pw_ref[0] + pw_ref[1] + 1e-16)
    rsel = (lax.broadcasted_iota(jnp.int32, (8, 16), 0)
            == (lax.broadcasted_iota(jnp.int32, (8, 16), 1) >> 1)
            ).astype(jnp.float32)
    inv16 = jnp.dot(inv, rsel, preferred_element_type=jnp.float32)
    zsi = acc[:, 0:16] * inv16
    zdi = acc[:, 16:32] * inv16
    lane = lax.broadcasted_iota(jnp.int32, (16, 2), 0)
    col = lax.broadcasted_iota(jnp.int32, (16, 2), 1)
    sel = (lane % 2 == col).astype(jnp.float32)
    xs2 = jnp.dot(zsi, sel, preferred_element_type=jnp.float32) + cs_ref[...]
    xd2 = jnp.dot(zdi, sel, preferred_element_type=jnp.float32) + cd_ref[...]
    a2s = jnp.dot(xs2, a2s_w_ref[...], preferred_element_type=jnp.float32)
    a2d = jnp.dot(xd2, a2d_w_ref[...], preferred_element_type=jnp.float32)
    m2 = _leaky(jnp.max(a2s, axis=0, keepdims=True) + a2d)
    ones = jnp.ones_like(a2s)
    zeros4 = jnp.zeros((xs2.shape[0], 4), jnp.float32)
    g2_ref[...] = jnp.concatenate([a2s, ones, xs2, zeros4], axis=1)
    d2_ref[...] = jnp.concatenate([a2d, m2, zeros4, jnp.zeros_like(xs2)], axis=1)


def _tc3_body(p2_ref, b2_ref, out_ref):
    acc2 = p2_ref[0] + p2_ref[1]
    denom = acc2[:, 0:1]
    num = acc2[:, 1:3]
    out_ref[...] = num / (denom + 1e-16) + b2_ref[...]


_MESH = plsc.VectorSubcoreMesh(core_axis_name="c", subcore_axis_name="s",
                               num_cores=NC, num_subcores=NS)


def _sc_edge_pipeline(compute_chunk):
    def body(idx_hbm, g_hbm, d_hbm, z_hbm, out_hbm,
             acc_sh,
             idx_a, idx_b, sd_a, sd_b,
             g_a, g_b, d_a, d_b, m_a, m_b,
             sg_a, sg_b, sdm_a, sdm_b, ss_a, ss_b):
        c = lax.axis_index("c")
        s = lax.axis_index("s")
        wid = s * NC + c
        row0 = s * ROWS_PER_S

        idxv = (idx_a, idx_b)
        sdst = (sd_a, sd_b)
        gbuf = (g_a, g_b)
        dbuf = (d_a, d_b)
        msg = (m_a, m_b)
        semg = (sg_a, sg_b)
        semd = (sdm_a, sdm_b)
        sems = (ss_a, ss_b)

        pltpu.sync_copy(z_hbm.at[pl.ds(row0, ROWS_PER_S)],
                        acc_sh.at[pl.ds(row0, ROWS_PER_S)])
        plsc.subcore_barrier()

        def fetch(j, b):
            pltpu.sync_copy(idx_hbm.at[pl.ds(wid * NCHUNK + j, 1)], idxv[b])
            pltpu.async_copy(g_hbm.at[idxv[b].at[0, pl.ds(0, CHUNK)]],
                             gbuf[b], semg[b])
            pltpu.async_copy(d_hbm.at[idxv[b].at[0, pl.ds(CHUNK, CHUNK)]],
                             dbuf[b], semd[b])

        def wait_gathers(b):
            pltpu.make_async_copy(g_hbm.at[idxv[b].at[0, pl.ds(0, CHUNK)]],
                                  gbuf[b], semg[b]).wait()
            pltpu.make_async_copy(d_hbm.at[idxv[b].at[0, pl.ds(CHUNK, CHUNK)]],
                                  dbuf[b], semd[b]).wait()

        def compute(b):
            for i in range(CHUNK // 16):
                sdst[b][pl.ds(16 * i, 16)] = idxv[b][0, pl.ds(CHUNK + 16 * i, 16)]
            compute_chunk(gbuf[b], dbuf[b], msg[b])

        def issue_scatter(b):
            pltpu.async_copy(msg[b], acc_sh.at[sdst[b]], sems[b], add=True)

        def wait_scatter(b):
            pltpu.make_async_copy(msg[b], acc_sh.at[sdst[b]], sems[b]).wait()

        fetch(0, 0)
        fetch(1, 1)
        for b in (0, 1):
            wait_gathers(b)
            compute(b)
            issue_scatter(b)
            fetch(b + 2, b)

        def pair(t, carry):
            for b in (0, 1):
                j = 2 * t + b
                wait_scatter(b)
                wait_gathers(b)
                compute(b)
                issue_scatter(b)

                @pl.when(j + 2 < NCHUNK)
                def _():
                    fetch(j + 2, b)
            return carry

        lax.fori_loop(1, (NCHUNK - 1) // 2, pair, 0)

        wait_scatter(0)
        wait_gathers(0)
        compute(0)
        issue_scatter(0)

        wait_scatter(1)
        wait_scatter(0)
        plsc.subcore_barrier()
        pltpu.sync_copy(acc_sh.at[pl.ds(row0, ROWS_PER_S)],
                        out_hbm.at[c, pl.ds(row0, ROWS_PER_S)])

    return body


def _sc_edge1(idx_hbm, g_hbm, d_hbm, z_hbm, zw_hbm, out_hbm, outw_hbm,
              acc_sh, accw_sh,
              idx_a, idx_b, sd_a, sd_b,
              g_a, g_b, d_a, d_b, m_a, m_b, w_a, w_b,
              sg_a, sg_b, sdm_a, sdm_b, sy_a, sy_b, sw_a, sw_b):
    c = lax.axis_index("c")
    s = lax.axis_index("s")
    wid = s * NC + c
    row0 = s * ROWS_PER_S

    idxv = (idx_a, idx_b)
    sdst = (sd_a, sd_b)
    gbuf = (g_a, g_b)
    dbuf = (d_a, d_b)
    msgy = (m_a, m_b)
    msgw = (w_a, w_b)
    semg = (sg_a, sg_b)
    semd = (sdm_a, sdm_b)
    semy = (sy_a, sy_b)
    semw = (sw_a, sw_b)

    pltpu.sync_copy(z_hbm.at[pl.ds(row0, ROWS_PER_S)],
                    acc_sh.at[pl.ds(row0, ROWS_PER_S)])
    pltpu.sync_copy(zw_hbm.at[pl.ds(row0, ROWS_PER_S)],
                    accw_sh.at[pl.ds(row0, ROWS_PER_S)])
    plsc.subcore_barrier()

    iota = lax.broadcasted_iota(jnp.int32, (16,), 0)
    idx_m = (iota & 7) + 8
    idx_w = iota >> 1
    col8 = iota & 7
    lt8 = iota < 8

    def fetch(j, b):
        pltpu.sync_copy(idx_hbm.at[pl.ds(wid * NCHUNK + j, 1)], idxv[b])
        pltpu.async_copy(g_hbm.at[idxv[b].at[0, pl.ds(0, CHUNK)]],
                         gbuf[b], semg[b])
        pltpu.async_copy(d_hbm.at[idxv[b].at[0, pl.ds(CHUNK, CHUNK)]],
                         dbuf[b], semd[b])

    def wait_gathers(b):
        pltpu.make_async_copy(g_hbm.at[idxv[b].at[0, pl.ds(0, CHUNK)]],
                              gbuf[b], semg[b]).wait()
        pltpu.make_async_copy(d_hbm.at[idxv[b].at[0, pl.ds(CHUNK, CHUNK)]],
                              dbuf[b], semd[b]).wait()

    def compute(b):
        for i in range(CHUNK // 16):
            sdst[b][pl.ds(16 * i, 16)] = idxv[b][0, pl.ds(CHUNK + 16 * i, 16)]

        def edge(e, carry):
            g0 = gbuf[b][e, pl.ds(0, 16)]
            d0 = dbuf[b][e, pl.ds(0, 16)]
            s0 = g0 + d0
            lk = _leaky(s0)
            mv = _perm(s0, idx_m)
            ex = jnp.exp(lk - mv)
            wexp = _perm(ex, idx_w)
            msgy[b][e, pl.ds(0, 16)] = wexp * gbuf[b][e, pl.ds(16, 16)]
            msgy[b][e, pl.ds(16, 16)] = wexp * gbuf[b][e, pl.ds(32, 16)]
            plsc.store_scatter(msgw[b], [iota * 0 + e, col8], ex, mask=lt8)
            return carry

        lax.fori_loop(0, CHUNK, edge, 0, unroll=8)

    def issue_scatter(b):
        pltpu.async_copy(msgy[b], acc_sh.at[sdst[b]], semy[b], add=True)
        pltpu.async_copy(msgw[b], accw_sh.at[sdst[b]], semw[b], add=True)

    def wait_scatter(b):
        pltpu.make_async_copy(msgy[b], acc_sh.at[sdst[b]], semy[b]).wait()
        pltpu.make_async_copy(msgw[b], accw_sh.at[sdst[b]], semw[b]).wait()

    fetch(0, 0)
    fetch(1, 1)
    for b in (0, 1):
        wait_gathers(b)
        compute(b)
        issue_scatter(b)
        fetch(b + 2, b)

    def pair(t, carry):
        for b in (0, 1):
            j = 2 * t + b
            wait_scatter(b)
            wait_gathers(b)
            compute(b)
            issue_scatter(b)

            @pl.when(j + 2 < NCHUNK)
            def _():
                fetch(j + 2, b)
        return carry

    lax.fori_loop(1, (NCHUNK - 1) // 2, pair, 0)

    wait_scatter(0)
    wait_gathers(0)
    compute(0)
    issue_scatter(0)

    wait_scatter(1)
    wait_scatter(0)
    plsc.subcore_barrier()
    pltpu.sync_copy(acc_sh.at[pl.ds(row0, ROWS_PER_S)],
                    out_hbm.at[c, pl.ds(row0, ROWS_PER_S)])
    pltpu.sync_copy(accw_sh.at[pl.ds(row0, ROWS_PER_S)],
                    outw_hbm.at[c, pl.ds(row0, ROWS_PER_S)])


def _compute_chunk2(gbuf, dbuf, msg):
    iota = lax.broadcasted_iota(jnp.int32, (16,), 0)
    half = iota >> 3
    coli = iota & 7
    idx_m = half * 8 + 1
    idx_w = half * 8
    lt3 = coli < 3
    idx_sh = jnp.where(lt3, half * 8 + coli + 1, half * 8)

    def pair(e2, carry):
        rowi = half + 2 * e2
        g2 = plsc.load_gather(gbuf, [rowi, coli])
        d2 = plsc.load_gather(dbuf, [rowi, coli])
        s0 = g2 + d2
        lk = _leaky(s0)
        mv = _perm(d2, idx_m)
        ex = jnp.exp(lk - mv)
        wv = _perm(ex, idx_w)
        gs = _perm(g2, idx_sh)
        plsc.store_scatter(msg, [rowi, coli], jnp.where(lt3, wv * gs, 0.0))
        return carry

    lax.fori_loop(0, CHUNK // 2, pair, 0, unroll=8)


_sc_edge2 = _sc_edge_pipeline(_compute_chunk2)


_SC_PARAMS = pltpu.CompilerParams(use_tc_tiling_on_sc=False,
                                  needs_layout_passes=False)

_sc1 = functools.partial(
    pl.kernel, _sc_edge1,
    out_type=[jax.ShapeDtypeStruct((NC, N, 32), jnp.float32),
              jax.ShapeDtypeStruct((NC, N, 8), jnp.float32)],
    mesh=_MESH,
    compiler_params=_SC_PARAMS,
    scratch_types=(
        [pltpu.VMEM_SHARED((N, 32), jnp.float32),
         pltpu.VMEM_SHARED((N, 8), jnp.float32)]
        + [pltpu.VMEM((1, 2 * CHUNK), jnp.int32)] * 2
        + [pltpu.VMEM((CHUNK,), jnp.int32)] * 2
        + [pltpu.VMEM((CHUNK, 48), jnp.float32)] * 2
        + [pltpu.VMEM((CHUNK, 16), jnp.float32)] * 2
        + [pltpu.VMEM((CHUNK, 32), jnp.float32)] * 2
        + [pltpu.VMEM((CHUNK, 8), jnp.float32)] * 2
        + [pltpu.SemaphoreType.DMA] * 8
    ),
)()

_sc2 = functools.partial(
    pl.kernel, _sc_edge2,
    out_type=jax.ShapeDtypeStruct((NC, N, 8), jnp.float32),
    mesh=_MESH,
    compiler_params=_SC_PARAMS,
    scratch_types=(
        [pltpu.VMEM_SHARED((N, 8), jnp.float32)]
        + [pltpu.VMEM((1, 2 * CHUNK), jnp.int32)] * 2
        + [pltpu.VMEM((CHUNK,), jnp.int32)] * 2
        + [pltpu.VMEM((CHUNK, 8), jnp.float32)] * 6
        + [pltpu.SemaphoreType.DMA] * 6
    ),
)()


@jax.jit
def kernel(x, edge_index, W1_src, W1_dst, att1_src, att1_dst, b1,
           W2_src, W2_dst, att2_src, att2_dst, b2):
    src = edge_index[0]
    dst = edge_index[1]
    idx_rows = jnp.concatenate([src.reshape(-1, CHUNK), dst.reshape(-1, CHUNK)],
                               axis=1)

    W1s3 = W1_src.reshape(F_IN, H1, HID)
    W1d3 = W1_dst.reshape(F_IN, H1, HID)
    Vs = jnp.einsum('fhc,hc->fh', W1s3, att1_src)
    Vd = jnp.einsum('fhc,hc->fh', W1d3, att1_dst)
    Us = jnp.einsum('fhc,hco->fho', W1s3, W2_src.reshape(H1, HID, OUT))
    Ud = jnp.einsum('fhc,hco->fho', W1s3, W2_dst.reshape(H1, HID, OUT))
    Bg = jnp.concatenate([Vs, jnp.zeros((F_IN, 8), jnp.float32),
                          Us.reshape(F_IN, 16), Ud.reshape(F_IN, 16)], axis=1)
    cs = (b1 @ W2_src).reshape(1, OUT)
    cd = (b1 @ W2_dst).reshape(1, OUT)

    gsrc, dtab = pl.pallas_call(
        _tc1_body,
        out_shape=[jax.ShapeDtypeStruct((N, 48), jnp.float32),
                   jax.ShapeDtypeStruct((N, 16), jnp.float32)],
    )(x, Bg, Vd)

    z32 = jnp.zeros((N, 32), jnp.float32)
    z8 = jnp.zeros((N, 8), jnp.float32)
    p1, pw1 = _sc1(idx_rows, gsrc, dtab, z32, z8)

    g2, d2 = pl.pallas_call(
        _tc2_body,
        out_shape=[jax.ShapeDtypeStruct((N, 8), jnp.float32),
                   jax.ShapeDtypeStruct((N, 8), jnp.float32)],
    )(p1, pw1, cs, cd,
      att2_src.reshape(OUT, 1), att2_dst.reshape(OUT, 1))

    p2 = _sc2(idx_rows, g2, d2, z8)

    out = pl.pallas_call(
        _tc3_body,
        out_shape=jax.ShapeDtypeStruct((N, OUT), jnp.float32),
    )(p2, b2.reshape(1, OUT))
    return out


# speedup vs baseline: 1.1914x; 1.0370x over previous
import functools

import jax
import jax.numpy as jnp
from jax import lax
from jax.experimental import pallas as pl
from jax.experimental.pallas import tpu as pltpu
from jax.experimental.pallas import tpu_sc as plsc

N = 10000
E = 320000
F_IN = 128
HID = 512
H1 = 8
OUT = 2

NC = 2
NS = 16
NW = NC * NS
EDGES_PER_W = E // NW
CHUNK = 80
NCHUNK = EDGES_PER_W // CHUNK
CHUNK1 = 200
HA1, HB1 = 104, 96
NCHUNK1 = EDGES_PER_W // CHUNK1
ROWS_PER_S = N // NS

_LEAK = 0.2


def _leaky(v):
    return jnp.maximum(v, _LEAK * v)


def _perm(v, idx):
    dnums = lax.GatherDimensionNumbers(
        offset_dims=(), collapsed_slice_dims=(0,), start_index_map=(0,))
    return lax.gather(v, idx[:, None], dnums, (1,),
                      mode=lax.GatherScatterMode.PROMISE_IN_BOUNDS)


def _tc1_body(x_ref, bg_ref, bd_ref, gs_ref, d_ref):
    xv = x_ref[...]
    gs = jnp.dot(xv, bg_ref[...], preferred_element_type=jnp.float32)
    gs_ref[...] = gs
    a1d = jnp.dot(xv, bd_ref[...], preferred_element_type=jnp.float32)
    a1s = gs[:, 0:8]
    gmax = jnp.max(a1s, axis=0, keepdims=True)
    m = gmax + a1d
    d_ref[...] = jnp.concatenate([a1d, _leaky(m)], axis=1)


def _tc2_body(p_ref, pw_ref, cs_ref, cd_ref, a2s_w_ref, a2d_w_ref,
              g2_ref, d2_ref):
    acc = p_ref[0] + p_ref[1]
    inv = 1.0 / (pw_ref[0] + pw_ref[1] + 1e-16)
    rsel = (lax.broadcasted_iota(jnp.int32, (8, 16), 0)
            == (lax.broadcasted_iota(jnp.int32, (8, 16), 1) >> 1)
            ).astype(jnp.float32)
    inv16 = jnp.dot(inv, rsel, preferred_element_type=jnp.float32)
    zsi = acc[:, 0:16] * inv16
    zdi = acc[:, 16:32] * inv16
    lane = lax.broadcasted_iota(jnp.int32, (16, 2), 0)
    col = lax.broadcasted_iota(jnp.int32, (16, 2), 1)
    sel = (lane % 2 == col).astype(jnp.float32)
    xs2 = jnp.dot(zsi, sel, preferred_element_type=jnp.float32) + cs_ref[...]
    xd2 = jnp.dot(zdi, sel, preferred_element_type=jnp.float32) + cd_ref[...]
    a2s = jnp.dot(xs2, a2s_w_ref[...], preferred_element_type=jnp.float32)
    a2d = jnp.dot(xd2, a2d_w_ref[...], preferred_element_type=jnp.float32)
    m2 = _leaky(jnp.max(a2s, axis=0, keepdims=True) + a2d)
    ones = jnp.ones_like(a2s)
    zeros4 = jnp.zeros((xs2.shape[0], 4), jnp.float32)
    g2_ref[...] = jnp.concatenate([a2s, ones, xs2, zeros4], axis=1)
    d2_ref[...] = jnp.concatenate([a2d, m2, zeros4, jnp.zeros_like(xs2)], axis=1)


def _tc3_body(p2_ref, b2_ref, out_ref):
    acc2 = p2_ref[0] + p2_ref[1]
    denom = acc2[:, 0:1]
    num = acc2[:, 1:3]
    out_ref[...] = num / (denom + 1e-16) + b2_ref[...]


_MESH = plsc.VectorSubcoreMesh(core_axis_name="c", subcore_axis_name="s",
                               num_cores=NC, num_subcores=NS)


def _sc_edge_pipeline(compute_chunk):
    def body(idx_hbm, g_hbm, d_hbm, z_hbm, out_hbm,
             acc_sh,
             idx_a, idx_b, sd_a, sd_b,
             g_a, g_b, d_a, d_b, m_a, m_b,
             sg_a, sg_b, sdm_a, sdm_b, ss_a, ss_b):
        c = lax.axis_index("c")
        s = lax.axis_index("s")
        wid = s * NC + c
        row0 = s * ROWS_PER_S

        idxv = (idx_a, idx_b)
        sdst = (sd_a, sd_b)
        gbuf = (g_a, g_b)
        dbuf = (d_a, d_b)
        msg = (m_a, m_b)
        semg = (sg_a, sg_b)
        semd = (sdm_a, sdm_b)
        sems = (ss_a, ss_b)

        pltpu.sync_copy(z_hbm.at[pl.ds(row0, ROWS_PER_S)],
                        acc_sh.at[pl.ds(row0, ROWS_PER_S)])
        plsc.subcore_barrier()

        def fetch(j, b):
            pltpu.sync_copy(idx_hbm.at[pl.ds(wid * NCHUNK + j, 1)], idxv[b])
            pltpu.async_copy(g_hbm.at[idxv[b].at[0, pl.ds(0, CHUNK)]],
                             gbuf[b], semg[b])
            pltpu.async_copy(d_hbm.at[idxv[b].at[0, pl.ds(CHUNK, CHUNK)]],
                             dbuf[b], semd[b])

        def wait_gathers(b):
            pltpu.make_async_copy(g_hbm.at[idxv[b].at[0, pl.ds(0, CHUNK)]],
                                  gbuf[b], semg[b]).wait()
            pltpu.make_async_copy(d_hbm.at[idxv[b].at[0, pl.ds(CHUNK, CHUNK)]],
                                  dbuf[b], semd[b]).wait()

        def compute(b):
            for i in range(CHUNK // 16):
                sdst[b][pl.ds(16 * i, 16)] = idxv[b][0, pl.ds(CHUNK + 16 * i, 16)]
            compute_chunk(gbuf[b], dbuf[b], msg[b])

        def issue_scatter(b):
            pltpu.async_copy(msg[b], acc_sh.at[sdst[b]], sems[b], add=True)

        def wait_scatter(b):
            pltpu.make_async_copy(msg[b], acc_sh.at[sdst[b]], sems[b]).wait()

        fetch(0, 0)
        fetch(1, 1)
        for b in (0, 1):
            wait_gathers(b)
            compute(b)
            issue_scatter(b)
            fetch(b + 2, b)

        def pair(t, carry):
            for b in (0, 1):
                j = 2 * t + b
                wait_scatter(b)
                wait_gathers(b)
                compute(b)
                issue_scatter(b)

                @pl.when(j + 2 < NCHUNK)
                def _():
                    fetch(j + 2, b)
            return carry

        lax.fori_loop(1, (NCHUNK - 1) // 2, pair, 0)

        wait_scatter(0)
        wait_gathers(0)
        compute(0)
        issue_scatter(0)

        wait_scatter(1)
        wait_scatter(0)
        plsc.subcore_barrier()
        pltpu.sync_copy(acc_sh.at[pl.ds(row0, ROWS_PER_S)],
                        out_hbm.at[c, pl.ds(row0, ROWS_PER_S)])

    return body


def _sc_edge1(idx_hbm, g_hbm, d_hbm, z_hbm, zw_hbm, out_hbm, outw_hbm,
              acc_sh, accw_sh,
              idx_a, idx_b, sdl_a, sdl_b, sdh_a, sdh_b,
              g_a, g_b, d_a, d_b, m_a, m_b, w_a, w_b,
              sg_a, sg_b, sdm_a, sdm_b, sy_a, sy_b, sw_a, sw_b):
    c = lax.axis_index("c")
    s = lax.axis_index("s")
    wid = s * NC + c
    row0 = s * ROWS_PER_S

    idxv = (idx_a, idx_b)
    sdl = (sdl_a, sdl_b)
    sdh = (sdh_a, sdh_b)
    gbuf = (g_a, g_b)
    dbuf = (d_a, d_b)
    msgy = (m_a, m_b)
    msgw = (w_a, w_b)
    semg = (sg_a, sg_b)
    semd = (sdm_a, sdm_b)
    semy = (sy_a, sy_b)
    semw = (sw_a, sw_b)

    pltpu.sync_copy(z_hbm.at[pl.ds(row0, ROWS_PER_S)],
                    acc_sh.at[pl.ds(row0, ROWS_PER_S)])
    pltpu.sync_copy(zw_hbm.at[pl.ds(row0, ROWS_PER_S)],
                    accw_sh.at[pl.ds(row0, ROWS_PER_S)])
    plsc.subcore_barrier()

    iota = lax.broadcasted_iota(jnp.int32, (16,), 0)
    idx_m = (iota & 7) + 8
    idx_w = iota >> 1
    col8 = iota & 7
    lt8 = iota < 8

    _SPLITS = ((0, HA1), (HA1, HB1))

    def fetch(j, b):
        pltpu.sync_copy(idx_hbm.at[pl.ds(wid * NCHUNK1 + j, 1)], idxv[b])
        for off, sz in _SPLITS:
            pltpu.async_copy(
                g_hbm.at[idxv[b].at[0, pl.ds(off, sz)]],
                gbuf[b].at[pl.ds(off, sz)], semg[b])
            pltpu.async_copy(
                d_hbm.at[idxv[b].at[0, pl.ds(CHUNK1 + off, sz)]],
                dbuf[b].at[pl.ds(off, sz)], semd[b])

    def wait_gathers(b):
        for off, sz in _SPLITS:
            pltpu.make_async_copy(
                g_hbm.at[idxv[b].at[0, pl.ds(off, sz)]],
                gbuf[b].at[pl.ds(off, sz)], semg[b]).wait()
            pltpu.make_async_copy(
                d_hbm.at[idxv[b].at[0, pl.ds(CHUNK1 + off, sz)]],
                dbuf[b].at[pl.ds(off, sz)], semd[b]).wait()

    def compute(b):
        for i in [0, 16, 32, 48, 64, 80, 88]:
            sdl[b][pl.ds(i, 16)] = idxv[b][0, pl.ds(CHUNK1 + i, 16)]
        for i in [0, 16, 32, 48, 64, 80]:
            sdh[b][pl.ds(i, 16)] = idxv[b][0, pl.ds(CHUNK1 + HA1 + i, 16)]

        def edge(e, carry):
            g0 = gbuf[b][e, pl.ds(0, 16)]
            d0 = dbuf[b][e, pl.ds(0, 16)]
            s0 = g0 + d0
            lk = _leaky(s0)
            mv = _perm(s0, idx_m)
            ex = jnp.exp(lk - mv)
            wexp = _perm(ex, idx_w)
            msgy[b][e, pl.ds(0, 16)] = wexp * gbuf[b][e, pl.ds(16, 16)]
            msgy[b][e, pl.ds(16, 16)] = wexp * gbuf[b][e, pl.ds(32, 16)]
            plsc.store_scatter(msgw[b], [iota * 0 + e, col8], ex, mask=lt8)
            return carry

        lax.fori_loop(0, CHUNK1, edge, 0, unroll=8)

    def issue_scatter(b):
        for (off, sz), sd in zip(_SPLITS, (sdl[b], sdh[b])):
            pltpu.async_copy(msgy[b].at[pl.ds(off, sz)],
                             acc_sh.at[sd], semy[b], add=True)
            pltpu.async_copy(msgw[b].at[pl.ds(off, sz)],
                             accw_sh.at[sd], semw[b], add=True)

    def wait_scatter(b):
        for (off, sz), sd in zip(_SPLITS, (sdl[b], sdh[b])):
            pltpu.make_async_copy(msgy[b].at[pl.ds(off, sz)],
                                  acc_sh.at[sd], semy[b]).wait()
            pltpu.make_async_copy(msgw[b].at[pl.ds(off, sz)],
                                  accw_sh.at[sd], semw[b]).wait()

    fetch(0, 0)
    fetch(1, 1)
    for b in (0, 1):
        wait_gathers(b)
        compute(b)
        issue_scatter(b)
        fetch(b + 2, b)

    def pair(t, carry):
        for b in (0, 1):
            j = 2 * t + b
            wait_scatter(b)
            wait_gathers(b)
            compute(b)
            issue_scatter(b)

            @pl.when(j + 2 < NCHUNK1)
            def _():
                fetch(j + 2, b)
        return carry

    lax.fori_loop(1, NCHUNK1 // 2, pair, 0)

    wait_scatter(1)
    wait_scatter(0)
    plsc.subcore_barrier()
    pltpu.sync_copy(acc_sh.at[pl.ds(row0, ROWS_PER_S)],
                    out_hbm.at[c, pl.ds(row0, ROWS_PER_S)])
    pltpu.sync_copy(accw_sh.at[pl.ds(row0, ROWS_PER_S)],
                    outw_hbm.at[c, pl.ds(row0, ROWS_PER_S)])


def _compute_chunk2(gbuf, dbuf, msg):
    iota = lax.broadcasted_iota(jnp.int32, (16,), 0)
    half = iota >> 3
    coli = iota & 7
    idx_m = half * 8 + 1
    idx_w = half * 8
    lt3 = coli < 3
    idx_sh = jnp.where(lt3, half * 8 + coli + 1, half * 8)

    def pair(e2, carry):
        rowi = half + 2 * e2
        g2 = plsc.load_gather(gbuf, [rowi, coli])
        d2 = plsc.load_gather(dbuf, [rowi, coli])
        s0 = g2 + d2
        lk = _leaky(s0)
        mv = _perm(d2, idx_m)
        ex = jnp.exp(lk - mv)
        wv = _perm(ex, idx_w)
        gs = _perm(g2, idx_sh)
        plsc.store_scatter(msg, [rowi, coli], jnp.where(lt3, wv * gs, 0.0))
        return carry

    lax.fori_loop(0, CHUNK // 2, pair, 0, unroll=8)


_sc_edge2 = _sc_edge_pipeline(_compute_chunk2)


_SC_PARAMS = pltpu.CompilerParams(use_tc_tiling_on_sc=False,
                                  needs_layout_passes=False)

_sc1 = functools.partial(
    pl.kernel, _sc_edge1,
    out_type=[jax.ShapeDtypeStruct((NC, N, 32), jnp.float32),
              jax.ShapeDtypeStruct((NC, N, 8), jnp.float32)],
    mesh=_MESH,
    compiler_params=_SC_PARAMS,
    scratch_types=(
        [pltpu.VMEM_SHARED((N, 32), jnp.float32),
         pltpu.VMEM_SHARED((N, 8), jnp.float32)]
        + [pltpu.VMEM((1, 2 * CHUNK1), jnp.int32)] * 2
        + [pltpu.VMEM((HA1,), jnp.int32)] * 2
        + [pltpu.VMEM((HB1,), jnp.int32)] * 2
        + [pltpu.VMEM((CHUNK1, 48), jnp.float32)] * 2
        + [pltpu.VMEM((CHUNK1, 16), jnp.float32)] * 2
        + [pltpu.VMEM((CHUNK1, 32), jnp.float32)] * 2
        + [pltpu.VMEM((CHUNK1, 8), jnp.float32)] * 2
        + [pltpu.SemaphoreType.DMA] * 8
    ),
)()

_sc2 = functools.partial(
    pl.kernel, _sc_edge2,
    out_type=jax.ShapeDtypeStruct((NC, N, 8), jnp.float32),
    mesh=_MESH,
    compiler_params=_SC_PARAMS,
    scratch_types=(
        [pltpu.VMEM_SHARED((N, 8), jnp.float32)]
        + [pltpu.VMEM((1, 2 * CHUNK), jnp.int32)] * 2
        + [pltpu.VMEM((CHUNK,), jnp.int32)] * 2
        + [pltpu.VMEM((CHUNK, 8), jnp.float32)] * 6
        + [pltpu.SemaphoreType.DMA] * 6
    ),
)()


@jax.jit
def kernel(x, edge_index, W1_src, W1_dst, att1_src, att1_dst, b1,
           W2_src, W2_dst, att2_src, att2_dst, b2):
    src = edge_index[0]
    dst = edge_index[1]
    idx_rows = jnp.concatenate([src.reshape(-1, CHUNK), dst.reshape(-1, CHUNK)],
                               axis=1)
    idx_rows1 = jnp.concatenate([src.reshape(-1, CHUNK1),
                                 dst.reshape(-1, CHUNK1)], axis=1)

    W1s3 = W1_src.reshape(F_IN, H1, HID)
    W1d3 = W1_dst.reshape(F_IN, H1, HID)
    Vs = jnp.einsum('fhc,hc->fh', W1s3, att1_src)
    Vd = jnp.einsum('fhc,hc->fh', W1d3, att1_dst)
    Us = jnp.einsum('fhc,hco->fho', W1s3, W2_src.reshape(H1, HID, OUT))
    Ud = jnp.einsum('fhc,hco->fho', W1s3, W2_dst.reshape(H1, HID, OUT))
    Bg = jnp.concatenate([Vs, jnp.zeros((F_IN, 8), jnp.float32),
                          Us.reshape(F_IN, 16), Ud.reshape(F_IN, 16)], axis=1)
    cs = (b1 @ W2_src).reshape(1, OUT)
    cd = (b1 @ W2_dst).reshape(1, OUT)

    gsrc, dtab = pl.pallas_call(
        _tc1_body,
        out_shape=[jax.ShapeDtypeStruct((N, 48), jnp.float32),
                   jax.ShapeDtypeStruct((N, 16), jnp.float32)],
    )(x, Bg, Vd)

    z32 = jnp.zeros((N, 32), jnp.float32)
    z8 = jnp.zeros((N, 8), jnp.float32)
    p1, pw1 = _sc1(idx_rows1, gsrc, dtab, z32, z8)

    g2, d2 = pl.pallas_call(
        _tc2_body,
        out_shape=[jax.ShapeDtypeStruct((N, 8), jnp.float32),
                   jax.ShapeDtypeStruct((N, 8), jnp.float32)],
    )(p1, pw1, cs, cd,
      att2_src.reshape(OUT, 1), att2_dst.reshape(OUT, 1))

    p2 = _sc2(idx_rows, g2, d2, z8)

    out = pl.pallas_call(
        _tc3_body,
        out_shape=jax.ShapeDtypeStruct((N, OUT), jnp.float32),
    )(p2, b2.reshape(1, OUT))
    return out


# speedup vs baseline: 1.2886x; 1.0816x over previous
import functools

import jax
import jax.numpy as jnp
from jax import lax
from jax.experimental import pallas as pl
from jax.experimental.pallas import tpu as pltpu
from jax.experimental.pallas import tpu_sc as plsc

N = 10000
E = 320000
F_IN = 128
HID = 512
H1 = 8
OUT = 2

NC = 2
NS = 16
NW = NC * NS
EDGES_PER_W = E // NW
CHUNK = 80
NCHUNK = EDGES_PER_W // CHUNK
CHUNK1 = 200
HA1, HB1 = 104, 96
NCHUNK1 = EDGES_PER_W // CHUNK1
ROWS_PER_S = N // NS

_LEAK = 0.2


def _leaky(v):
    return jnp.maximum(v, _LEAK * v)


def _perm(v, idx):
    dnums = lax.GatherDimensionNumbers(
        offset_dims=(), collapsed_slice_dims=(0,), start_index_map=(0,))
    return lax.gather(v, idx[:, None], dnums, (1,),
                      mode=lax.GatherScatterMode.PROMISE_IN_BOUNDS)


def _tc1_body(x_ref, bg_ref, bd_ref, gs_ref, d_ref):
    xv = x_ref[...]
    gs = jnp.dot(xv, bg_ref[...], preferred_element_type=jnp.float32)
    gs_ref[...] = gs
    a1d = jnp.dot(xv, bd_ref[...], preferred_element_type=jnp.float32)
    a1s = gs[:, 0:8]
    gmax = jnp.max(a1s, axis=0, keepdims=True)
    m = gmax + a1d
    d_ref[...] = jnp.concatenate([a1d, _leaky(m)], axis=1)


def _tc2_body(p_ref, pw_ref, cs_ref, cd_ref, a2s_w_ref, a2d_w_ref,
              g2_ref, d2_ref):
    acc = p_ref[0] + p_ref[1]
    inv = 1.0 / (pw_ref[0] + pw_ref[1] + 1e-16)
    rsel = (lax.broadcasted_iota(jnp.int32, (8, 16), 0)
            == (lax.broadcasted_iota(jnp.int32, (8, 16), 1) >> 1)
            ).astype(jnp.float32)
    inv16 = jnp.dot(inv, rsel, preferred_element_type=jnp.float32)
    zsi = acc[:, 0:16] * inv16
    zdi = acc[:, 16:32] * inv16
    lane = lax.broadcasted_iota(jnp.int32, (16, 2), 0)
    col = lax.broadcasted_iota(jnp.int32, (16, 2), 1)
    sel = (lane % 2 == col).astype(jnp.float32)
    xs2 = jnp.dot(zsi, sel, preferred_element_type=jnp.float32) + cs_ref[...]
    xd2 = jnp.dot(zdi, sel, preferred_element_type=jnp.float32) + cd_ref[...]
    a2s = jnp.dot(xs2, a2s_w_ref[...], preferred_element_type=jnp.float32)
    a2d = jnp.dot(xd2, a2d_w_ref[...], preferred_element_type=jnp.float32)
    m2 = _leaky(jnp.max(a2s, axis=0, keepdims=True) + a2d)
    ones = jnp.ones_like(a2s)
    zeros4 = jnp.zeros((xs2.shape[0], 4), jnp.float32)
    g2_ref[...] = jnp.concatenate([a2s, ones, xs2, zeros4], axis=1)
    d2_ref[...] = jnp.concatenate([a2d, m2, zeros4, jnp.zeros_like(xs2)], axis=1)


def _tc3_body(p2_ref, b2_ref, out_ref):
    acc2 = p2_ref[0] + p2_ref[1]
    denom = acc2[:, 0:1]
    num = acc2[:, 1:3]
    out_ref[...] = num / (denom + 1e-16) + b2_ref[...]


_MESH = plsc.VectorSubcoreMesh(core_axis_name="c", subcore_axis_name="s",
                               num_cores=NC, num_subcores=NS)


def _sc_edge_pipeline(compute_chunk):
    def body(idx_hbm, g_hbm, d_hbm, z_hbm, out_hbm,
             acc_sh,
             idx_a, idx_b, sdl_a, sdl_b, sdh_a, sdh_b,
             g_a, g_b, d_a, d_b, m_a, m_b,
             sg_a, sg_b, sdm_a, sdm_b, ss_a, ss_b):
        c = lax.axis_index("c")
        s = lax.axis_index("s")
        wid = s * NC + c
        row0 = s * ROWS_PER_S

        idxv = (idx_a, idx_b)
        sdl = (sdl_a, sdl_b)
        sdh = (sdh_a, sdh_b)
        gbuf = (g_a, g_b)
        dbuf = (d_a, d_b)
        msg = (m_a, m_b)
        semg = (sg_a, sg_b)
        semd = (sdm_a, sdm_b)
        sems = (ss_a, ss_b)

        pltpu.sync_copy(z_hbm.at[pl.ds(row0, ROWS_PER_S)],
                        acc_sh.at[pl.ds(row0, ROWS_PER_S)])
        plsc.subcore_barrier()

        _SPLITS = ((0, HA1), (HA1, HB1))

        def fetch(j, b):
            pltpu.sync_copy(idx_hbm.at[pl.ds(wid * NCHUNK1 + j, 1)], idxv[b])
            for off, sz in _SPLITS:
                pltpu.async_copy(g_hbm.at[idxv[b].at[0, pl.ds(off, sz)]],
                                 gbuf[b].at[pl.ds(off, sz)], semg[b])
                pltpu.async_copy(d_hbm.at[idxv[b].at[0, pl.ds(CHUNK1 + off, sz)]],
                                 dbuf[b].at[pl.ds(off, sz)], semd[b])

        def wait_gathers(b):
            for off, sz in _SPLITS:
                pltpu.make_async_copy(g_hbm.at[idxv[b].at[0, pl.ds(off, sz)]],
                                      gbuf[b].at[pl.ds(off, sz)], semg[b]).wait()
                pltpu.make_async_copy(d_hbm.at[idxv[b].at[0, pl.ds(CHUNK1 + off, sz)]],
                                      dbuf[b].at[pl.ds(off, sz)], semd[b]).wait()

        def compute(b):
            for i in [0, 16, 32, 48, 64, 80, 88]:
                sdl[b][pl.ds(i, 16)] = idxv[b][0, pl.ds(CHUNK1 + i, 16)]
            for i in [0, 16, 32, 48, 64, 80]:
                sdh[b][pl.ds(i, 16)] = idxv[b][0, pl.ds(CHUNK1 + HA1 + i, 16)]
            compute_chunk(gbuf[b], dbuf[b], msg[b])

        def issue_scatter(b):
            for (off, sz), sd in zip(_SPLITS, (sdl[b], sdh[b])):
                pltpu.async_copy(msg[b].at[pl.ds(off, sz)],
                                 acc_sh.at[sd], sems[b], add=True)

        def wait_scatter(b):
            for (off, sz), sd in zip(_SPLITS, (sdl[b], sdh[b])):
                pltpu.make_async_copy(msg[b].at[pl.ds(off, sz)],
                                      acc_sh.at[sd], sems[b]).wait()

        fetch(0, 0)
        fetch(1, 1)
        for b in (0, 1):
            wait_gathers(b)
            compute(b)
            issue_scatter(b)
            fetch(b + 2, b)

        def pair(t, carry):
            for b in (0, 1):
                j = 2 * t + b
                wait_scatter(b)
                wait_gathers(b)
                compute(b)
                issue_scatter(b)

                @pl.when(j + 2 < NCHUNK1)
                def _():
                    fetch(j + 2, b)
            return carry

        lax.fori_loop(1, NCHUNK1 // 2, pair, 0)

        wait_scatter(1)
        wait_scatter(0)
        plsc.subcore_barrier()
        pltpu.sync_copy(acc_sh.at[pl.ds(row0, ROWS_PER_S)],
                        out_hbm.at[c, pl.ds(row0, ROWS_PER_S)])

    return body


def _sc_edge1(idx_hbm, g_hbm, d_hbm, z_hbm, zw_hbm, out_hbm, outw_hbm,
              acc_sh, accw_sh,
              idx_a, idx_b, sdl_a, sdl_b, sdh_a, sdh_b,
              g_a, g_b, d_a, d_b, m_a, m_b, w_a, w_b,
              sg_a, sg_b, sdm_a, sdm_b, sy_a, sy_b, sw_a, sw_b):
    c = lax.axis_index("c")
    s = lax.axis_index("s")
    wid = s * NC + c
    row0 = s * ROWS_PER_S

    idxv = (idx_a, idx_b)
    sdl = (sdl_a, sdl_b)
    sdh = (sdh_a, sdh_b)
    gbuf = (g_a, g_b)
    dbuf = (d_a, d_b)
    msgy = (m_a, m_b)
    msgw = (w_a, w_b)
    semg = (sg_a, sg_b)
    semd = (sdm_a, sdm_b)
    semy = (sy_a, sy_b)
    semw = (sw_a, sw_b)

    pltpu.sync_copy(z_hbm.at[pl.ds(row0, ROWS_PER_S)],
                    acc_sh.at[pl.ds(row0, ROWS_PER_S)])
    pltpu.sync_copy(zw_hbm.at[pl.ds(row0, ROWS_PER_S)],
                    accw_sh.at[pl.ds(row0, ROWS_PER_S)])
    plsc.subcore_barrier()

    iota = lax.broadcasted_iota(jnp.int32, (16,), 0)
    idx_m = (iota & 7) + 8
    idx_w = iota >> 1
    col8 = iota & 7
    lt8 = iota < 8

    _SPLITS = ((0, HA1), (HA1, HB1))

    def fetch(j, b):
        pltpu.sync_copy(idx_hbm.at[pl.ds(wid * NCHUNK1 + j, 1)], idxv[b])
        for off, sz in _SPLITS:
            pltpu.async_copy(
                g_hbm.at[idxv[b].at[0, pl.ds(off, sz)]],
                gbuf[b].at[pl.ds(off, sz)], semg[b])
            pltpu.async_copy(
                d_hbm.at[idxv[b].at[0, pl.ds(CHUNK1 + off, sz)]],
                dbuf[b].at[pl.ds(off, sz)], semd[b])

    def wait_gathers(b):
        for off, sz in _SPLITS:
            pltpu.make_async_copy(
                g_hbm.at[idxv[b].at[0, pl.ds(off, sz)]],
                gbuf[b].at[pl.ds(off, sz)], semg[b]).wait()
            pltpu.make_async_copy(
                d_hbm.at[idxv[b].at[0, pl.ds(CHUNK1 + off, sz)]],
                dbuf[b].at[pl.ds(off, sz)], semd[b]).wait()

    def compute(b):
        for i in [0, 16, 32, 48, 64, 80, 88]:
            sdl[b][pl.ds(i, 16)] = idxv[b][0, pl.ds(CHUNK1 + i, 16)]
        for i in [0, 16, 32, 48, 64, 80]:
            sdh[b][pl.ds(i, 16)] = idxv[b][0, pl.ds(CHUNK1 + HA1 + i, 16)]

        def edge(e, carry):
            g0 = gbuf[b][e, pl.ds(0, 16)]
            d0 = dbuf[b][e, pl.ds(0, 16)]
            s0 = g0 + d0
            lk = _leaky(s0)
            mv = _perm(s0, idx_m)
            ex = jnp.exp(lk - mv)
            wexp = _perm(ex, idx_w)
            msgy[b][e, pl.ds(0, 16)] = wexp * gbuf[b][e, pl.ds(16, 16)]
            msgy[b][e, pl.ds(16, 16)] = wexp * gbuf[b][e, pl.ds(32, 16)]
            plsc.store_scatter(msgw[b], [iota * 0 + e, col8], ex, mask=lt8)
            return carry

        lax.fori_loop(0, CHUNK1, edge, 0, unroll=8)

    def issue_scatter(b):
        for (off, sz), sd in zip(_SPLITS, (sdl[b], sdh[b])):
            pltpu.async_copy(msgy[b].at[pl.ds(off, sz)],
                             acc_sh.at[sd], semy[b], add=True)
            pltpu.async_copy(msgw[b].at[pl.ds(off, sz)],
                             accw_sh.at[sd], semw[b], add=True)

    def wait_scatter(b):
        for (off, sz), sd in zip(_SPLITS, (sdl[b], sdh[b])):
            pltpu.make_async_copy(msgy[b].at[pl.ds(off, sz)],
                                  acc_sh.at[sd], semy[b]).wait()
            pltpu.make_async_copy(msgw[b].at[pl.ds(off, sz)],
                                  accw_sh.at[sd], semw[b]).wait()

    fetch(0, 0)
    fetch(1, 1)
    for b in (0, 1):
        wait_gathers(b)
        compute(b)
        issue_scatter(b)
        fetch(b + 2, b)

    def pair(t, carry):
        for b in (0, 1):
            j = 2 * t + b
            wait_scatter(b)
            wait_gathers(b)
            compute(b)
            issue_scatter(b)

            @pl.when(j + 2 < NCHUNK1)
            def _():
                fetch(j + 2, b)
        return carry

    lax.fori_loop(1, NCHUNK1 // 2, pair, 0)

    wait_scatter(1)
    wait_scatter(0)
    plsc.subcore_barrier()
    pltpu.sync_copy(acc_sh.at[pl.ds(row0, ROWS_PER_S)],
                    out_hbm.at[c, pl.ds(row0, ROWS_PER_S)])
    pltpu.sync_copy(accw_sh.at[pl.ds(row0, ROWS_PER_S)],
                    outw_hbm.at[c, pl.ds(row0, ROWS_PER_S)])


def _compute_chunk2(gbuf, dbuf, msg):
    iota = lax.broadcasted_iota(jnp.int32, (16,), 0)
    half = iota >> 3
    coli = iota & 7
    idx_m = half * 8 + 1
    idx_w = half * 8
    lt3 = coli < 3
    idx_sh = jnp.where(lt3, half * 8 + coli + 1, half * 8)

    def pair(e2, carry):
        rowi = half + 2 * e2
        g2 = plsc.load_gather(gbuf, [rowi, coli])
        d2 = plsc.load_gather(dbuf, [rowi, coli])
        s0 = g2 + d2
        lk = _leaky(s0)
        mv = _perm(d2, idx_m)
        ex = jnp.exp(lk - mv)
        wv = _perm(ex, idx_w)
        gs = _perm(g2, idx_sh)
        plsc.store_scatter(msg, [rowi, coli], jnp.where(lt3, wv * gs, 0.0))
        return carry

    lax.fori_loop(0, CHUNK1 // 2, pair, 0, unroll=8)


_sc_edge2 = _sc_edge_pipeline(_compute_chunk2)


_SC_PARAMS = pltpu.CompilerParams(use_tc_tiling_on_sc=False,
                                  needs_layout_passes=False)

_sc1 = functools.partial(
    pl.kernel, _sc_edge1,
    out_type=[jax.ShapeDtypeStruct((NC, N, 32), jnp.float32),
              jax.ShapeDtypeStruct((NC, N, 8), jnp.float32)],
    mesh=_MESH,
    compiler_params=_SC_PARAMS,
    scratch_types=(
        [pltpu.VMEM_SHARED((N, 32), jnp.float32),
         pltpu.VMEM_SHARED((N, 8), jnp.float32)]
        + [pltpu.VMEM((1, 2 * CHUNK1), jnp.int32)] * 2
        + [pltpu.VMEM((HA1,), jnp.int32)] * 2
        + [pltpu.VMEM((HB1,), jnp.int32)] * 2
        + [pltpu.VMEM((CHUNK1, 48), jnp.float32)] * 2
        + [pltpu.VMEM((CHUNK1, 16), jnp.float32)] * 2
        + [pltpu.VMEM((CHUNK1, 32), jnp.float32)] * 2
        + [pltpu.VMEM((CHUNK1, 8), jnp.float32)] * 2
        + [pltpu.SemaphoreType.DMA] * 8
    ),
)()

_sc2 = functools.partial(
    pl.kernel, _sc_edge2,
    out_type=jax.ShapeDtypeStruct((NC, N, 8), jnp.float32),
    mesh=_MESH,
    compiler_params=_SC_PARAMS,
    scratch_types=(
        [pltpu.VMEM_SHARED((N, 8), jnp.float32)]
        + [pltpu.VMEM((1, 2 * CHUNK1), jnp.int32)] * 2
        + [pltpu.VMEM((HA1,), jnp.int32)] * 2
        + [pltpu.VMEM((HB1,), jnp.int32)] * 2
        + [pltpu.VMEM((CHUNK1, 8), jnp.float32)] * 6
        + [pltpu.SemaphoreType.DMA] * 6
    ),
)()


@jax.jit
def kernel(x, edge_index, W1_src, W1_dst, att1_src, att1_dst, b1,
           W2_src, W2_dst, att2_src, att2_dst, b2):
    src = edge_index[0]
    dst = edge_index[1]
    idx_rows1 = jnp.concatenate([src.reshape(-1, CHUNK1),
                                 dst.reshape(-1, CHUNK1)], axis=1)

    W1s3 = W1_src.reshape(F_IN, H1, HID)
    W1d3 = W1_dst.reshape(F_IN, H1, HID)
    Vs = jnp.einsum('fhc,hc->fh', W1s3, att1_src)
    Vd = jnp.einsum('fhc,hc->fh', W1d3, att1_dst)
    Us = jnp.einsum('fhc,hco->fho', W1s3, W2_src.reshape(H1, HID, OUT))
    Ud = jnp.einsum('fhc,hco->fho', W1s3, W2_dst.reshape(H1, HID, OUT))
    Bg = jnp.concatenate([Vs, jnp.zeros((F_IN, 8), jnp.float32),
                          Us.reshape(F_IN, 16), Ud.reshape(F_IN, 16)], axis=1)
    cs = (b1 @ W2_src).reshape(1, OUT)
    cd = (b1 @ W2_dst).reshape(1, OUT)

    gsrc, dtab = pl.pallas_call(
        _tc1_body,
        out_shape=[jax.ShapeDtypeStruct((N, 48), jnp.float32),
                   jax.ShapeDtypeStruct((N, 16), jnp.float32)],
    )(x, Bg, Vd)

    z32 = jnp.zeros((N, 32), jnp.float32)
    z8 = jnp.zeros((N, 8), jnp.float32)
    p1, pw1 = _sc1(idx_rows1, gsrc, dtab, z32, z8)

    g2, d2 = pl.pallas_call(
        _tc2_body,
        out_shape=[jax.ShapeDtypeStruct((N, 8), jnp.float32),
                   jax.ShapeDtypeStruct((N, 8), jnp.float32)],
    )(p1, pw1, cs, cd,
      att2_src.reshape(OUT, 1), att2_dst.reshape(OUT, 1))

    p2 = _sc2(idx_rows1, g2, d2, z8)

    out = pl.pallas_call(
        _tc3_body,
        out_shape=jax.ShapeDtypeStruct((N, OUT), jnp.float32),
    )(p2, b2.reshape(1, OUT))
    return out
